# baseline (speedup 1.0000x reference)
import jax
import jax.numpy as jnp
from jax import lax
from jax.experimental import pallas as pl
from jax.experimental.pallas import tpu as pltpu
from jax.experimental.pallas import tpu_sc as plsc

N, S, H, W, D = 8, 32, 14, 14, 8
E = 64
NC, NS = 2, 16


def _sc_body(table_hbm, out_hbm, buf, row_buf, sem):
    s = lax.axis_index("s") * NC + lax.axis_index("c")

    t0 = (s // 8) * 8
    pltpu.sync_copy(table_hbm.at[pl.ds(t0, 8)], row_buf)
    r = s - t0

    regs = [row_buf[r, pl.ds(16 * j, 16)] for j in range(4)]

    def rep(i, carry):
        h = i // W
        w = lax.rem(i, W)
        for d in range(D):
            for j in range(4):
                buf[h, w, d, pl.ds(16 * j, 16)] = regs[j]
        return carry

    lax.fori_loop(0, (H // 2) * W, rep, 0)

    copies = [
        pltpu.async_copy(buf, out_hbm.at[n, s, pl.ds(half * (H // 2), H // 2)], sem)
        for n in range(N)
        for half in range(2)
    ]
    for c in copies:
        c.wait()


@jax.jit
def _sc_embed(table):
    mesh = plsc.VectorSubcoreMesh(core_axis_name="c", subcore_axis_name="s")
    k = pl.kernel(
        _sc_body,
        mesh=mesh,
        out_type=jax.ShapeDtypeStruct((N, S, H, W, D, E), jnp.float32),
        scratch_types=[
            pltpu.VMEM((H // 2, W, D, E), jnp.float32),
            pltpu.VMEM((8, E), jnp.float32),
            pltpu.SemaphoreType.DMA,
        ],
    )
    return k(table)


def kernel(x, table):
    del x
    return _sc_embed(table)


# speedup vs baseline: 1.0011x; 1.0011x over previous
import jax
import jax.numpy as jnp
from jax import lax
from jax.experimental import pallas as pl
from jax.experimental.pallas import tpu as pltpu
from jax.experimental.pallas import tpu_sc as plsc

N, S, H, W, D = 8, 32, 14, 14, 8
E = 64
NC, NS = 2, 16


def _sc_body(table_hbm, out_hbm, buf, row_buf, sem):
    s = lax.axis_index("s") * NC + lax.axis_index("c")

    t0 = (s // 8) * 8
    pltpu.sync_copy(table_hbm.at[pl.ds(t0, 8)], row_buf)
    r = s - t0

    regs = [row_buf[r, pl.ds(16 * j, 16)] for j in range(4)]

    def rep(i, carry):
        h = i // W
        w = lax.rem(i, W)
        for d in range(D):
            for j in range(4):
                buf[h, w, d, pl.ds(16 * j, 16)] = regs[j]
        return carry

    lax.fori_loop(0, (H // 2) * W, rep, 0)

    copies = [
        pltpu.async_copy(buf, out_hbm.at[n, s, pl.ds(half * (H // 2), H // 2)], sem)
        for n in range(N)
        for half in range(2)
    ]
    for c in copies:
        c.wait()


@jax.jit
def _sc_embed(table):
    mesh = plsc.VectorSubcoreMesh(core_axis_name="c", subcore_axis_name="s")
    k = pl.kernel(
        _sc_body,
        mesh=mesh,
        out_type=jax.ShapeDtypeStruct((N, S, H, W, D, E), jnp.float32),
        scratch_types=[
            pltpu.VMEM((H // 2, W, D, E), jnp.float32),
            pltpu.VMEM((8, E), jnp.float32),
            pltpu.SemaphoreType.DMA,
        ],
    )
    return k(table[:S])


def kernel(x, table):
    del x
    return _sc_embed(table)
